# baseline (speedup 1.0000x reference)
import functools

import jax
import jax.numpy as jnp
from jax import lax
from jax.experimental import pallas as pl
from jax.experimental.pallas import tpu as pltpu
from jax.experimental.pallas import tpu_sc as plsc

N = 10000
FEAT = 128
HID = 128
LAT = 64
EPS = 1e-5

NSC = 2
NSUB = 16
NT = NSC * NSUB
CH = 64
NPAD = 10240
ROWS_PER_TILE = NPAD // NSUB
DEGW = 128
CORE0_FRAC = 0.5


def _sc_mesh():
    return plsc.VectorSubcoreMesh(core_axis_name="c", subcore_axis_name="s")


def _unpack_dst(packed_v, j, dbuf):
    for t in range(CH // 16):
        v = packed_v[j, pl.ds(16 * t, 16)]
        dbuf[pl.ds(16 * t, 16)] = jax.lax.shift_right_logical(v, 16)


def _unpack_src(packed_v, j, sbuf):
    for t in range(CH // 16):
        v = packed_v[j, pl.ds(16 * t, 16)]
        sbuf[pl.ds(16 * t, 16)] = jax.lax.bitwise_and(v, 0xFFFF)


def _sc_degree(packed_f, ones_b, zeros_b, nch):
    @functools.partial(
        pl.kernel,
        out_type=jax.ShapeDtypeStruct((NSC, NPAD, DEGW), jnp.float32),
        mesh=_sc_mesh(),
        scratch_types=[
            pltpu.VMEM((nch, CH), jnp.int32),
            pltpu.VMEM((CH,), jnp.int32),
            pltpu.VMEM((CH,), jnp.int32),
            pltpu.VMEM((CH, DEGW), jnp.float32),
            pltpu.VMEM((CH, DEGW), jnp.float32),
            pltpu.VMEM_SHARED((NPAD, DEGW), jnp.float32),
            pltpu.SemaphoreType.DMA,
            pltpu.SemaphoreType.DMA,
        ],
    )
    def k(packed_hbm, ones_hbm, zeros_hbm, out_hbm, packed_v, da_v, db_v,
          ones_v, stage_v, acc_sh, sem0, sem1):
        c = lax.axis_index("c")
        s = lax.axis_index("s")
        wid = c * NSUB + s
        start = pl.multiple_of(wid * nch, 8)
        pltpu.sync_copy(packed_hbm.at[pl.ds(start, nch)], packed_v)
        pltpu.sync_copy(ones_hbm, ones_v)
        pltpu.sync_copy(zeros_hbm, stage_v)
        row0 = s * ROWS_PER_TILE
        for k2 in range(ROWS_PER_TILE // CH):
            pltpu.sync_copy(stage_v, acc_sh.at[pl.ds(row0 + k2 * CH, CH)])
        plsc.subcore_barrier()

        def sc_start(dbuf, sem):
            pltpu.async_copy(ones_v, acc_sh.at[dbuf], sem, add=True)

        def sc_wait(dbuf, sem):
            pltpu.make_async_copy(ones_v, acc_sh.at[dbuf], sem).wait()

        _unpack_dst(packed_v, 0, da_v)
        sc_start(da_v, sem0)
        _unpack_dst(packed_v, 1, db_v)
        sc_start(db_v, sem1)

        def body(i, carry):
            j0 = 2 * i
            sc_wait(da_v, sem0)
            _unpack_dst(packed_v, j0, da_v)
            sc_start(da_v, sem0)
            sc_wait(db_v, sem1)
            _unpack_dst(packed_v, j0 + 1, db_v)
            sc_start(db_v, sem1)
            return carry

        lax.fori_loop(1, nch // 2, body, 0)
        sc_wait(da_v, sem0)
        sc_wait(db_v, sem1)
        plsc.subcore_barrier()
        for k2 in range(ROWS_PER_TILE // CH):
            r = row0 + k2 * CH
            pltpu.sync_copy(acc_sh.at[pl.ds(r, CH)], stage_v)
            pltpu.sync_copy(stage_v, out_hbm.at[c, pl.ds(r, CH)])

    return k(packed_f, ones_b, zeros_b)


def _sc_scatter(hp, packed_f, zeros_b, nch0, nch1):
    nchmax = max(nch0, nch1)

    @functools.partial(
        pl.kernel,
        out_type=jax.ShapeDtypeStruct((NSC, NPAD, FEAT), jnp.float32),
        mesh=_sc_mesh(),
        scratch_types=[
            pltpu.VMEM((nchmax, CH), jnp.int32),
            pltpu.VMEM((CH,), jnp.int32),
            pltpu.VMEM((CH,), jnp.int32),
            pltpu.VMEM((CH,), jnp.int32),
            pltpu.VMEM((CH,), jnp.int32),
            pltpu.VMEM((CH, FEAT), jnp.float32),
            pltpu.VMEM((CH, FEAT), jnp.float32),
            pltpu.VMEM_SHARED((NPAD, FEAT), jnp.float32),
            pltpu.SemaphoreType.DMA,
            pltpu.SemaphoreType.DMA,
        ],
    )
    def k(hp_hbm, packed_hbm, zeros_hbm, out_hbm, packed_v,
          sa_v, sb_v, da_v, db_v, rows0_v, rows1_v, acc_sh,
          gsem0, gsem1):
        c = lax.axis_index("c")
        s = lax.axis_index("s")

        @pl.when(c == 0)
        def _():
            start = pl.multiple_of(s * nch0, 8)
            pltpu.sync_copy(packed_hbm.at[pl.ds(start, nch0)],
                            packed_v.at[pl.ds(0, nch0)])

        @pl.when(c == 1)
        def _():
            start = pl.multiple_of(NSUB * nch0 + s * nch1, 8)
            pltpu.sync_copy(packed_hbm.at[pl.ds(start, nch1)],
                            packed_v.at[pl.ds(0, nch1)])

        pltpu.sync_copy(zeros_hbm, rows0_v)
        row0 = s * ROWS_PER_TILE
        for k2 in range(ROWS_PER_TILE // CH):
            pltpu.sync_copy(rows0_v, acc_sh.at[pl.ds(row0 + k2 * CH, CH)])
        plsc.subcore_barrier()

        def cvt(j, sbuf, dbuf):
            _unpack_src(packed_v, j, sbuf)
            _unpack_dst(packed_v, j, dbuf)

        def g(sbuf, rbuf, sem):
            return pltpu.make_async_copy(hp_hbm.at[sbuf], rbuf, sem)

        def run(nch_c):
            cvt(0, sa_v, da_v)
            g(sa_v, rows0_v, gsem0).start()

            def body(i, carry):
                j0 = 2 * i
                cvt(j0 + 1, sb_v, db_v)
                g(sb_v, rows1_v, gsem1).start()
                g(sa_v, rows0_v, gsem0).wait()
                pltpu.sync_copy(rows0_v, acc_sh.at[da_v], add=True)

                @pl.when(i < nch_c // 2 - 1)
                def _():
                    cvt(j0 + 2, sa_v, da_v)
                    g(sa_v, rows0_v, gsem0).start()

                g(sb_v, rows1_v, gsem1).wait()
                pltpu.sync_copy(rows1_v, acc_sh.at[db_v], add=True)
                return carry

            lax.fori_loop(0, nch_c // 2, body, 0)

        @pl.when(c == 0)
        def _():
            run(nch0)

        @pl.when(c == 1)
        def _():
            run(nch1)

        plsc.subcore_barrier()
        for k2 in range(ROWS_PER_TILE // CH):
            r = row0 + k2 * CH
            pltpu.sync_copy(acc_sh.at[pl.ds(r, CH)], rows0_v)
            pltpu.sync_copy(rows0_v, out_hbm.at[c, pl.ds(r, CH)])

    return k(hp, packed_f, zeros_b)


def _tc_a1_body(x_ref, w_ref, xw_ref):
    xw_ref[...] = jnp.dot(x_ref[...], w_ref[...],
                          preferred_element_type=jnp.float32)


def _tc_a_body(xw_ref, deg_ref, hp_ref, dinv_ref):
    deg = deg_ref[:, 0:1] + deg_ref[:, 1:2] + 1.0
    rows = lax.broadcasted_iota(jnp.int32, (NPAD, 1), 0)
    dinv = jnp.where(rows < N, lax.rsqrt(deg), 0.0)
    dinv_ref[...] = dinv
    hp_ref[0:N, :] = xw_ref[...] * dinv[0:N]
    hp_ref[N:NPAD, :] = jnp.zeros((NPAD - N, HID), jnp.float32)


def _bn_relu_cond(s0, s1, hp, dinv, b, g, be, hwc, hb):
    agg = (s0 + s1 + hp) * dinv + b
    rows = lax.broadcasted_iota(jnp.int32, (NPAD, 1), 0)
    valid = rows < N
    aggm = jnp.where(valid, agg, 0.0)
    mean = jnp.sum(aggm, axis=0, keepdims=True) * (1.0 / N)
    cent = agg - mean
    var = jnp.sum(jnp.where(valid, cent * cent, 0.0), axis=0,
                  keepdims=True) * (1.0 / N)
    hn = cent * lax.rsqrt(var + EPS) * g + be
    hom = jnp.sum(hwc, axis=0, keepdims=True) + hb
    return jnp.maximum(hn, 0.0) + hom


def _tc_b_body(s_ref, hp_ref, dinv_ref, b_ref, g_ref, be_ref, hwc_ref,
               hb_ref, w1_ref, hp1_ref):
    dinv = dinv_ref[...]
    h = _bn_relu_cond(s_ref[0], s_ref[1], hp_ref[...], dinv, b_ref[...],
                      g_ref[...], be_ref[...], hwc_ref[...], hb_ref[...])
    hp1_ref[...] = jnp.dot(h, w1_ref[...],
                           preferred_element_type=jnp.float32) * dinv


def _tc_c_body(s_ref, hp_ref, dinv_ref, b_ref, g_ref, be_ref, hwc_ref,
               hb_ref, muwh_ref, mucc_ref, mub_ref, lvwh_ref, lvcc_ref,
               lvb_ref, mu_ref, lv_ref):
    h = _bn_relu_cond(s_ref[0], s_ref[1], hp_ref[...], dinv_ref[...],
                      b_ref[...], g_ref[...], be_ref[...], hwc_ref[...],
                      hb_ref[...])
    mu_c = jnp.sum(mucc_ref[...], axis=0, keepdims=True) + mub_ref[...]
    lv_c = jnp.sum(lvcc_ref[...], axis=0, keepdims=True) + lvb_ref[...]
    hv = h[0:N, :]
    mu_ref[...] = jnp.dot(hv, muwh_ref[...],
                          preferred_element_type=jnp.float32) + mu_c
    lv_ref[...] = jnp.dot(hv, lvwh_ref[...],
                          preferred_element_type=jnp.float32) + lv_c


def _tc_a(x, W0, deg2):
    xw = pl.pallas_call(
        _tc_a1_body,
        out_shape=jax.ShapeDtypeStruct((N, HID), jnp.float32),
    )(x, W0)
    return pl.pallas_call(
        _tc_a_body,
        out_shape=[jax.ShapeDtypeStruct((NPAD, HID), jnp.float32),
                   jax.ShapeDtypeStruct((NPAD, 1), jnp.float32)],
    )(xw, deg2)


def _tc_b(S, hp0, dinv, b0, g0, beta0, hwc0, hb0, W1):
    return pl.pallas_call(
        _tc_b_body,
        out_shape=jax.ShapeDtypeStruct((NPAD, HID), jnp.float32),
    )(S, hp0, dinv, b0, g0, beta0, hwc0, hb0, W1)


def _tc_c(S, hp1, dinv, b1, g1, beta1, hwc1, hb1, muWh, mucc, mub, lvWh,
          lvcc, lvb):
    return pl.pallas_call(
        _tc_c_body,
        out_shape=[jax.ShapeDtypeStruct((N, LAT), jnp.float32),
                   jax.ShapeDtypeStruct((N, LAT), jnp.float32)],
    )(S, hp1, dinv, b1, g1, beta1, hwc1, hb1, muWh, mucc, mub, lvWh,
      lvcc, lvb)


def kernel(x, edge_index, homophily_cond, W0, b0, g0, beta0, hW0, hb0,
           W1, b1, g1, beta1, hW1, hb1, muW, mub, lvW, lvb):
    n, feat = x.shape
    e = edge_index.shape[1]
    tot = -(-e // (NSUB * CH))
    tot = -(-tot // 16) * 16
    nch0 = max(8, int(round(tot * CORE0_FRAC / 8)) * 8)
    nch1 = tot - nch0
    nch_sym = tot // 2
    epad = NSUB * tot * CH

    packed = jnp.bitwise_or(edge_index[0],
                            jnp.left_shift(edge_index[1], 16))
    spread = (jnp.arange(epad - e, dtype=jnp.int32) % (NPAD - N)) + N
    pad = jnp.bitwise_or(spread, jnp.left_shift(spread, 16))
    packed_f = jnp.concatenate([packed, pad]).reshape(NSUB * tot, CH)

    ones_b = jnp.ones((CH, DEGW), jnp.float32)
    zeros_deg = jnp.zeros((CH, DEGW), jnp.float32)
    zeros_rows = jnp.zeros((CH, FEAT), jnp.float32)

    deg_parts = _sc_degree(packed_f, ones_b, zeros_deg, nch_sym)
    deg2 = jnp.transpose(deg_parts[:, :, 0])
    hp0, dinv = _tc_a(x, W0, deg2)

    hc = homophily_cond[:, None]
    hwc0 = jnp.broadcast_to(hc, hW0.shape) * hW0
    hwc1 = jnp.broadcast_to(hc, hW1.shape) * hW1
    mucc = jnp.broadcast_to(hc, (3, LAT)) * muW[HID:]
    lvcc = jnp.broadcast_to(hc, (3, LAT)) * lvW[HID:]

    r2 = lambda v: v.reshape(1, -1)

    S0 = _sc_scatter(hp0, packed_f, zeros_rows, nch0, nch1)
    hp1 = _tc_b(S0, hp0, dinv, r2(b0), r2(g0), r2(beta0), hwc0, r2(hb0), W1)

    S1 = _sc_scatter(hp1, packed_f, zeros_rows, nch0, nch1)
    mu, lv = _tc_c(S1, hp1, dinv, r2(b1), r2(g1), r2(beta1), hwc1, r2(hb1),
                   muW[:HID], mucc, r2(mub), lvW[:HID], lvcc, r2(lvb))
    return (mu, lv)


# speedup vs baseline: 1.0025x; 1.0025x over previous
import functools

import jax
import jax.numpy as jnp
from jax import lax
from jax.experimental import pallas as pl
from jax.experimental.pallas import tpu as pltpu
from jax.experimental.pallas import tpu_sc as plsc

N = 10000
FEAT = 128
HID = 128
LAT = 64
EPS = 1e-5

NSC = 2
NSUB = 16
NT = NSC * NSUB
CH = 64
NPAD = 10240
ROWS_PER_TILE = NPAD // NSUB
DEGW = 128
CORE0_FRAC = 0.5


def _sc_mesh():
    return plsc.VectorSubcoreMesh(core_axis_name="c", subcore_axis_name="s")


def _unpack_dst(packed_v, j, dbuf):
    for t in range(CH // 16):
        v = packed_v[j, pl.ds(16 * t, 16)]
        dbuf[pl.ds(16 * t, 16)] = jax.lax.shift_right_logical(v, 16)


def _unpack_src(packed_v, j, sbuf):
    for t in range(CH // 16):
        v = packed_v[j, pl.ds(16 * t, 16)]
        sbuf[pl.ds(16 * t, 16)] = jax.lax.bitwise_and(v, 0xFFFF)


def _sc_degree(packed_f, ones_b, zeros_b, nch):
    @functools.partial(
        pl.kernel,
        out_type=jax.ShapeDtypeStruct((NSC, NPAD, DEGW), jnp.float32),
        mesh=_sc_mesh(),
        scratch_types=[
            pltpu.VMEM((nch, CH), jnp.int32),
            pltpu.VMEM((CH,), jnp.int32),
            pltpu.VMEM((CH,), jnp.int32),
            pltpu.VMEM((CH, DEGW), jnp.float32),
            pltpu.VMEM((CH, DEGW), jnp.float32),
            pltpu.VMEM_SHARED((NPAD, DEGW), jnp.float32),
            pltpu.SemaphoreType.DMA,
            pltpu.SemaphoreType.DMA,
        ],
    )
    def k(packed_hbm, ones_hbm, zeros_hbm, out_hbm, packed_v, da_v, db_v,
          ones_v, stage_v, acc_sh, sem0, sem1):
        c = lax.axis_index("c")
        s = lax.axis_index("s")
        wid = c * NSUB + s
        start = pl.multiple_of(wid * nch, 8)
        pltpu.sync_copy(packed_hbm.at[pl.ds(start, nch)], packed_v)
        pltpu.sync_copy(ones_hbm, ones_v)
        pltpu.sync_copy(zeros_hbm, stage_v)
        row0 = s * ROWS_PER_TILE
        for k2 in range(ROWS_PER_TILE // CH):
            pltpu.sync_copy(stage_v, acc_sh.at[pl.ds(row0 + k2 * CH, CH)])
        plsc.subcore_barrier()

        def sc_start(dbuf, sem):
            pltpu.async_copy(ones_v, acc_sh.at[dbuf], sem, add=True)

        def sc_wait(dbuf, sem):
            pltpu.make_async_copy(ones_v, acc_sh.at[dbuf], sem).wait()

        _unpack_dst(packed_v, 0, da_v)
        sc_start(da_v, sem0)
        _unpack_dst(packed_v, 1, db_v)
        sc_start(db_v, sem1)

        def body(i, carry):
            j0 = 2 * i
            sc_wait(da_v, sem0)
            _unpack_dst(packed_v, j0, da_v)
            sc_start(da_v, sem0)
            sc_wait(db_v, sem1)
            _unpack_dst(packed_v, j0 + 1, db_v)
            sc_start(db_v, sem1)
            return carry

        lax.fori_loop(1, nch // 2, body, 0)
        sc_wait(da_v, sem0)
        sc_wait(db_v, sem1)
        plsc.subcore_barrier()
        for k2 in range(ROWS_PER_TILE // CH):
            r = row0 + k2 * CH
            pltpu.sync_copy(acc_sh.at[pl.ds(r, CH)], stage_v)
            pltpu.sync_copy(stage_v, out_hbm.at[c, pl.ds(r, CH)])

    return k(packed_f, ones_b, zeros_b)


def _sc_scatter(hp, packed_f, zeros_b, nch0, nch1):
    nchmax = max(nch0, nch1)

    @functools.partial(
        pl.kernel,
        out_type=jax.ShapeDtypeStruct((NSC, NPAD, FEAT), jnp.float32),
        mesh=_sc_mesh(),
        scratch_types=[
            pltpu.VMEM((nchmax, CH), jnp.int32),
            pltpu.VMEM((CH,), jnp.int32),
            pltpu.VMEM((CH,), jnp.int32),
            pltpu.VMEM((CH,), jnp.int32),
            pltpu.VMEM((CH,), jnp.int32),
            pltpu.VMEM((CH, FEAT), jnp.float32),
            pltpu.VMEM((CH, FEAT), jnp.float32),
            pltpu.VMEM_SHARED((NPAD, FEAT), jnp.float32),
            pltpu.SemaphoreType.DMA,
            pltpu.SemaphoreType.DMA,
        ],
    )
    def k(hp_hbm, packed_hbm, zeros_hbm, out_hbm, packed_v,
          sa_v, sb_v, da_v, db_v, rows0_v, rows1_v, acc_sh,
          gsem0, gsem1):
        c = lax.axis_index("c")
        s = lax.axis_index("s")

        @pl.when(c == 0)
        def _():
            start = pl.multiple_of(s * nch0, 8)
            pltpu.sync_copy(packed_hbm.at[pl.ds(start, nch0)],
                            packed_v.at[pl.ds(0, nch0)])

        @pl.when(c == 1)
        def _():
            start = pl.multiple_of(NSUB * nch0 + s * nch1, 8)
            pltpu.sync_copy(packed_hbm.at[pl.ds(start, nch1)],
                            packed_v.at[pl.ds(0, nch1)])

        pltpu.sync_copy(zeros_hbm, rows0_v)
        row0 = s * ROWS_PER_TILE
        for k2 in range(ROWS_PER_TILE // CH):
            pltpu.sync_copy(rows0_v, acc_sh.at[pl.ds(row0 + k2 * CH, CH)])
        plsc.subcore_barrier()

        def cvt(j, sbuf, dbuf):
            _unpack_src(packed_v, j, sbuf)
            _unpack_dst(packed_v, j, dbuf)

        def g(sbuf, rbuf, sem):
            return pltpu.make_async_copy(hp_hbm.at[sbuf], rbuf, sem)

        def run(nch_c):
            cvt(0, sa_v, da_v)
            g(sa_v, rows0_v, gsem0).start()

            def body(i, carry):
                j0 = 2 * i
                cvt(j0 + 1, sb_v, db_v)
                g(sb_v, rows1_v, gsem1).start()
                g(sa_v, rows0_v, gsem0).wait()
                pltpu.sync_copy(rows0_v, acc_sh.at[da_v], add=True)

                @pl.when(i < nch_c // 2 - 1)
                def _():
                    cvt(j0 + 2, sa_v, da_v)
                    g(sa_v, rows0_v, gsem0).start()

                g(sb_v, rows1_v, gsem1).wait()
                pltpu.sync_copy(rows1_v, acc_sh.at[db_v], add=True)
                return carry

            lax.fori_loop(0, nch_c // 2, body, 0)

        @pl.when(c == 0)
        def _():
            run(nch0)

        @pl.when(c == 1)
        def _():
            run(nch1)

        plsc.subcore_barrier()
        for k2 in range(ROWS_PER_TILE // CH):
            r = row0 + k2 * CH
            pltpu.sync_copy(acc_sh.at[pl.ds(r, CH)], rows0_v)
            pltpu.sync_copy(rows0_v, out_hbm.at[c, pl.ds(r, CH)])

    return k(hp, packed_f, zeros_b)


def _tc_a_body(x_ref, w_ref, deg_ref, hp_ref, dinv_ref):
    deg = deg_ref[:, 0:1] + deg_ref[:, 1:2] + 1.0
    rows = lax.broadcasted_iota(jnp.int32, (NPAD, 1), 0)
    dinv = jnp.where(rows < N, lax.rsqrt(deg), 0.0)
    dinv_ref[...] = dinv
    xw = jnp.dot(x_ref[...], w_ref[...], preferred_element_type=jnp.float32)
    hp_ref[0:N, :] = xw * dinv[0:N]
    hp_ref[N:NPAD, :] = jnp.zeros((NPAD - N, HID), jnp.float32)


def _bn_relu_cond(s0, s1, hp, dinv, b, g, be, hwc, hb):
    agg = (s0 + s1 + hp) * dinv + b
    rows = lax.broadcasted_iota(jnp.int32, (NPAD, 1), 0)
    valid = rows < N
    aggm = jnp.where(valid, agg, 0.0)
    mean = jnp.sum(aggm, axis=0, keepdims=True) * (1.0 / N)
    cent = agg - mean
    var = jnp.sum(jnp.where(valid, cent * cent, 0.0), axis=0,
                  keepdims=True) * (1.0 / N)
    hn = cent * lax.rsqrt(var + EPS) * g + be
    hom = jnp.sum(hwc, axis=0, keepdims=True) + hb
    return jnp.maximum(hn, 0.0) + hom


def _tc_b_body(s_ref, hp_ref, dinv_ref, b_ref, g_ref, be_ref, hwc_ref,
               hb_ref, w1_ref, hp1_ref):
    dinv = dinv_ref[...]
    h = _bn_relu_cond(s_ref[0], s_ref[1], hp_ref[...], dinv, b_ref[...],
                      g_ref[...], be_ref[...], hwc_ref[...], hb_ref[...])
    hp1_ref[...] = jnp.dot(h, w1_ref[...],
                           preferred_element_type=jnp.float32) * dinv


def _tc_c_body(s_ref, hp_ref, dinv_ref, b_ref, g_ref, be_ref, hwc_ref,
               hb_ref, muwh_ref, mucc_ref, mub_ref, lvwh_ref, lvcc_ref,
               lvb_ref, mu_ref, lv_ref):
    h = _bn_relu_cond(s_ref[0], s_ref[1], hp_ref[...], dinv_ref[...],
                      b_ref[...], g_ref[...], be_ref[...], hwc_ref[...],
                      hb_ref[...])
    mu_c = jnp.sum(mucc_ref[...], axis=0, keepdims=True) + mub_ref[...]
    lv_c = jnp.sum(lvcc_ref[...], axis=0, keepdims=True) + lvb_ref[...]
    hv = h[0:N, :]
    mu_ref[...] = jnp.dot(hv, muwh_ref[...],
                          preferred_element_type=jnp.float32) + mu_c
    lv_ref[...] = jnp.dot(hv, lvwh_ref[...],
                          preferred_element_type=jnp.float32) + lv_c


def _tc_a(x, W0, deg2):
    return pl.pallas_call(
        _tc_a_body,
        out_shape=[jax.ShapeDtypeStruct((NPAD, HID), jnp.float32),
                   jax.ShapeDtypeStruct((NPAD, 1), jnp.float32)],
    )(x, W0, deg2)


def _tc_b(S, hp0, dinv, b0, g0, beta0, hwc0, hb0, W1):
    return pl.pallas_call(
        _tc_b_body,
        out_shape=jax.ShapeDtypeStruct((NPAD, HID), jnp.float32),
    )(S, hp0, dinv, b0, g0, beta0, hwc0, hb0, W1)


def _tc_c(S, hp1, dinv, b1, g1, beta1, hwc1, hb1, muWh, mucc, mub, lvWh,
          lvcc, lvb):
    return pl.pallas_call(
        _tc_c_body,
        out_shape=[jax.ShapeDtypeStruct((N, LAT), jnp.float32),
                   jax.ShapeDtypeStruct((N, LAT), jnp.float32)],
    )(S, hp1, dinv, b1, g1, beta1, hwc1, hb1, muWh, mucc, mub, lvWh,
      lvcc, lvb)


def kernel(x, edge_index, homophily_cond, W0, b0, g0, beta0, hW0, hb0,
           W1, b1, g1, beta1, hW1, hb1, muW, mub, lvW, lvb):
    n, feat = x.shape
    e = edge_index.shape[1]
    tot = -(-e // (NSUB * CH))
    tot = -(-tot // 16) * 16
    nch0 = max(8, int(round(tot * CORE0_FRAC / 8)) * 8)
    nch1 = tot - nch0
    nch_sym = tot // 2
    epad = NSUB * tot * CH

    packed = jnp.bitwise_or(edge_index[0],
                            jnp.left_shift(edge_index[1], 16))
    spread = (jnp.arange(epad - e, dtype=jnp.int32) % (NPAD - N)) + N
    pad = jnp.bitwise_or(spread, jnp.left_shift(spread, 16))
    packed_f = jnp.concatenate([packed, pad]).reshape(NSUB * tot, CH)

    ones_b = jnp.ones((CH, DEGW), jnp.float32)
    zeros_deg = jnp.zeros((CH, DEGW), jnp.float32)
    zeros_rows = jnp.zeros((CH, FEAT), jnp.float32)

    deg_parts = _sc_degree(packed_f, ones_b, zeros_deg, nch_sym)
    deg2 = jnp.transpose(deg_parts[:, :, 0])
    hp0, dinv = _tc_a(x, W0, deg2)

    hc = homophily_cond[:, None]
    hwc0 = jnp.broadcast_to(hc, hW0.shape) * hW0
    hwc1 = jnp.broadcast_to(hc, hW1.shape) * hW1
    mucc = jnp.broadcast_to(hc, (3, LAT)) * muW[HID:]
    lvcc = jnp.broadcast_to(hc, (3, LAT)) * lvW[HID:]

    r2 = lambda v: v.reshape(1, -1)

    S0 = _sc_scatter(hp0, packed_f, zeros_rows, nch0, nch1)
    hp1 = _tc_b(S0, hp0, dinv, r2(b0), r2(g0), r2(beta0), hwc0, r2(hb0), W1)

    S1 = _sc_scatter(hp1, packed_f, zeros_rows, nch0, nch1)
    mu, lv = _tc_c(S1, hp1, dinv, r2(b1), r2(g1), r2(beta1), hwc1, r2(hb1),
                   muW[:HID], mucc, r2(mub), lvW[:HID], lvcc, r2(lvb))
    return (mu, lv)
